# baseline (speedup 1.0000x reference)
import jax
import jax.numpy as jnp
from jax.experimental import pallas as pl
from jax.experimental.pallas import tpu as pltpu

_XCH = 8192
_ZCH = 1024
_KX = 3
_KZ = 3


def _pipe(src, dst, buf, sem_in, sem_out, nrows, chunk, depth, patch=None):
    n = nrows // chunk
    in_c = [None] * n
    out_c = [None] * n
    for k in range(n):
        j = k % depth
        if k >= depth:
            out_c[k - depth].wait()
        in_c[k] = pltpu.make_async_copy(
            src.at[pl.ds(k * chunk, chunk), :], buf.at[j], sem_in.at[j])
        in_c[k].start()
        if k >= 1:
            p = k - 1
            in_c[p].wait()
            if p == 0 and patch is not None:
                patch(buf.at[p % depth])
            out_c[p] = pltpu.make_async_copy(
                buf.at[p % depth], dst.at[pl.ds(p * chunk, chunk), :],
                sem_out.at[p % depth])
            out_c[p].start()
    p = n - 1
    in_c[p].wait()
    if p == 0 and patch is not None:
        patch(buf.at[p % depth])
    out_c[p] = pltpu.make_async_copy(
        buf.at[p % depth], dst.at[pl.ds(p * chunk, chunk), :],
        sem_out.at[p % depth])
    out_c[p].start()
    return out_c[max(0, n - depth):]


def _fused_kernel(x_hbm, y_vmem, z_hbm, w_smem, xo_hbm, zo_hbm,
                  xbuf, zbuf, sem_xi, sem_xo, sem_zi, sem_zo):
    def patch_x(b):
        xb = b[0:16, :]
        rows = jax.lax.broadcasted_iota(jnp.int32, xb.shape, 0)
        xb = jnp.where(rows == 10, y_vmem[0:1, :], xb)
        xb = jnp.where(rows == 2, y_vmem[1:2, :], xb)
        xb = jnp.where(rows == 1, jnp.float32(45.0), xb)
        b[0:16, :] = xb

    def patch_z(b):
        zb = b[0:8, :]
        rows = jax.lax.broadcasted_iota(jnp.int32, zb.shape, 0)
        cols = jax.lax.broadcasted_iota(jnp.int32, zb.shape, 1)
        upd = jnp.where((rows == 1) & (cols == 3), w_smem[0], 0.0)
        upd = jnp.where((rows == 0) & (cols == 2), w_smem[1], upd)
        upd = jnp.where((rows == 0) & (cols == 1), w_smem[2], upd)
        b[0:8, :] = zb + upd

    pending_x = _pipe(x_hbm, xo_hbm, xbuf, sem_xi, sem_xo,
                      x_hbm.shape[0], _XCH, _KX, patch_x)
    pending_z = _pipe(z_hbm, zo_hbm, zbuf, sem_zi, sem_zo,
                      z_hbm.shape[0], _ZCH, _KZ, patch_z)
    for c in pending_x + pending_z:
        c.wait()


def kernel(x, y, z, w):
    return pl.pallas_call(
        _fused_kernel,
        in_specs=[
            pl.BlockSpec(memory_space=pl.ANY),
            pl.BlockSpec(memory_space=pltpu.VMEM),
            pl.BlockSpec(memory_space=pl.ANY),
            pl.BlockSpec(memory_space=pltpu.SMEM),
        ],
        out_specs=[
            pl.BlockSpec(memory_space=pl.ANY),
            pl.BlockSpec(memory_space=pl.ANY),
        ],
        out_shape=[
            jax.ShapeDtypeStruct(x.shape, x.dtype),
            jax.ShapeDtypeStruct(z.shape, z.dtype),
        ],
        scratch_shapes=[
            pltpu.VMEM((_KX, _XCH, x.shape[1]), jnp.float32),
            pltpu.VMEM((_KZ, _ZCH, z.shape[1]), jnp.float32),
            pltpu.SemaphoreType.DMA((_KX,)),
            pltpu.SemaphoreType.DMA((_KX,)),
            pltpu.SemaphoreType.DMA((_KZ,)),
            pltpu.SemaphoreType.DMA((_KZ,)),
        ],
    )(x, y, z, w)
